# Initial kernel scaffold; baseline (speedup 1.0000x reference)
import jax
import jax.numpy as jnp
from jax.experimental import pallas as pl


def kernel(positions, features, global_features, W_g, b_g, W_l, b_l):
    raise NotImplementedError("write your pallas kernel here")


# speedup vs baseline: 1.0552x; 1.0552x over previous
import jax
import jax.numpy as jnp
from jax.experimental import pallas as pl

_RADIUS2 = 64.0
_TI = 256


def _global_body(pos_ref, feat_ref, gf_ref, wgp_ref, wgf_ref, wgg_ref, bg_ref,
                 wlg_ref, bl_ref, gout_ref, gterm_ref):
    agg_p = jnp.max(pos_ref[...], axis=2)
    agg_f = jnp.max(feat_ref[...], axis=2)
    g_lin = (jnp.dot(agg_p, wgp_ref[...], preferred_element_type=jnp.float32)
             + jnp.dot(agg_f, wgf_ref[...], preferred_element_type=jnp.float32)
             + jnp.dot(gf_ref[...], wgg_ref[...], preferred_element_type=jnp.float32)
             + bg_ref[...])
    g_out = jnp.tanh(g_lin)
    gout_ref[...] = g_out
    G = wlg_ref.shape[0]
    gu = g_out[:, G:]
    gterm_ref[...] = (jnp.dot(gu, wlg_ref[...], preferred_element_type=jnp.float32)
                      + bl_ref[...])


def _local_body(pos_ref, posr_ref, nodes_ref, nodesr_ref, gterm_ref,
                wla_ref, wlb_ref, out_ref):
    n = pos_ref.shape[2]
    ti = posr_ref.shape[2]
    xs = pos_ref[0, 0, :].reshape(1, n)
    ys = pos_ref[0, 1, :].reshape(1, n)
    zs = pos_ref[0, 2, :].reshape(1, n)
    xr = posr_ref[0, 0, :].reshape(ti, 1)
    yr = posr_ref[0, 1, :].reshape(ti, 1)
    zr = posr_ref[0, 2, :].reshape(ti, 1)
    dx = xr - xs
    dy = yr - ys
    dz = zr - zs
    d2 = dx * dx + dy * dy + dz * dz
    mask = (d2 < _RADIUS2).astype(jnp.float32)
    cnt = jnp.maximum(jnp.sum(mask, axis=1, keepdims=True), 1.0)
    nsum = jnp.dot(mask, nodes_ref[0], preferred_element_type=jnp.float32)
    nmean = nsum / cnt
    rows = nodesr_ref[0]
    lin = (jnp.dot(rows, wla_ref[...], preferred_element_type=jnp.float32)
           + jnp.dot(nmean, wlb_ref[...], preferred_element_type=jnp.float32)
           + gterm_ref[0])
    out_ref[0] = jnp.tanh(lin)


def kernel(positions, features, global_features, W_g, b_g, W_l, b_l):
    B, P, N = positions.shape
    F = features.shape[1]
    G = global_features.shape[1]
    C = P + F
    G2 = 2 * G

    wgp = W_g[:P]
    wgf = W_g[P:C]
    wgg = W_g[C:]
    wla = W_l[:C]
    wlb = W_l[C:2 * C]
    wlg = W_l[2 * C:]
    bg2 = b_g.reshape(1, G2)
    bl2 = b_l.reshape(1, C)

    g_out, gterm = pl.pallas_call(
        _global_body,
        out_shape=(
            jax.ShapeDtypeStruct((B, G2), jnp.float32),
            jax.ShapeDtypeStruct((B, C), jnp.float32),
        ),
    )(positions, features, global_features, wgp, wgf, wgg, bg2, wlg, bl2)

    nodes = jnp.concatenate([positions, features], axis=1).transpose(0, 2, 1)

    grid = (B, N // _TI)
    local_out = pl.pallas_call(
        _local_body,
        grid=grid,
        in_specs=[
            pl.BlockSpec((1, P, N), lambda b, i: (b, 0, 0)),
            pl.BlockSpec((1, P, _TI), lambda b, i: (b, 0, i)),
            pl.BlockSpec((1, N, C), lambda b, i: (b, 0, 0)),
            pl.BlockSpec((1, _TI, C), lambda b, i: (b, i, 0)),
            pl.BlockSpec((1, 1, C), lambda b, i: (b, 0, 0)),
            pl.BlockSpec((C, C), lambda b, i: (0, 0)),
            pl.BlockSpec((C, C), lambda b, i: (0, 0)),
        ],
        out_specs=pl.BlockSpec((1, _TI, C), lambda b, i: (b, i, 0)),
        out_shape=jax.ShapeDtypeStruct((B, N, C), jnp.float32),
    )(positions, positions, nodes, nodes, gterm.reshape(B, 1, C), wla, wlb)

    positions_new = local_out[:, :, :P].transpose(0, 2, 1)
    features_new = local_out[:, :, P:].transpose(0, 2, 1)
    global_new = g_out[:, :G]
    return (positions_new, features_new, global_new)


# speedup vs baseline: 1.2297x; 1.1654x over previous
import jax
import jax.numpy as jnp
from jax.experimental import pallas as pl

_RADIUS2 = 64.0
_TI = 512


def _global_body(pos_ref, feat_ref, gf_ref, wgp_ref, wgf_ref, wgg_ref, bg_ref,
                 wlg_ref, bl_ref, gout_ref, gterm_ref):
    agg_p = jnp.max(pos_ref[...], axis=2)
    agg_f = jnp.max(feat_ref[...], axis=2)
    g_lin = (jnp.dot(agg_p, wgp_ref[...], preferred_element_type=jnp.float32)
             + jnp.dot(agg_f, wgf_ref[...], preferred_element_type=jnp.float32)
             + jnp.dot(gf_ref[...], wgg_ref[...], preferred_element_type=jnp.float32)
             + bg_ref[...])
    g_out = jnp.tanh(g_lin)
    gout_ref[...] = g_out
    G = wlg_ref.shape[0]
    gu = g_out[:, G:]
    gterm_ref[...] = (jnp.dot(gu, wlg_ref[...], preferred_element_type=jnp.float32)
                      + bl_ref[...])


def _local_body(pos_ref, rows_ref, nodes_ref, nodesr_ref, gterm_ref,
                wla_ref, wlb_ref, out_ref):
    cols = pos_ref[0]
    rows = rows_ref[0]
    c = wla_ref.shape[0]
    rn_c = jnp.sum(cols * cols, axis=0, keepdims=True)
    rn_r = jnp.sum(rows * rows, axis=1, keepdims=True)
    thresh = 0.5 * (rn_r - _RADIUS2) + 0.5 * rn_c
    dot = jnp.dot(rows, cols, preferred_element_type=jnp.float32)
    mask = (dot > thresh).astype(jnp.float32)
    nsum = jnp.dot(mask, nodes_ref[0], preferred_element_type=jnp.float32)
    cnt = jnp.maximum(nsum[:, c:c + 1], 1.0)
    nmean = nsum[:, :c] / cnt
    lin = (jnp.dot(nodesr_ref[0], wla_ref[...], preferred_element_type=jnp.float32)
           + jnp.dot(nmean, wlb_ref[...], preferred_element_type=jnp.float32)
           + gterm_ref[0])
    out_ref[0] = jnp.tanh(lin)


def kernel(positions, features, global_features, W_g, b_g, W_l, b_l):
    B, P, N = positions.shape
    F = features.shape[1]
    G = global_features.shape[1]
    C = P + F
    G2 = 2 * G

    wgp = W_g[:P]
    wgf = W_g[P:C]
    wgg = W_g[C:]
    wla = W_l[:C]
    wlb = W_l[C:2 * C]
    wlg = W_l[2 * C:]
    bg2 = b_g.reshape(1, G2)
    bl2 = b_l.reshape(1, C)

    g_out, gterm = pl.pallas_call(
        _global_body,
        out_shape=(
            jax.ShapeDtypeStruct((B, G2), jnp.float32),
            jax.ShapeDtypeStruct((B, C), jnp.float32),
        ),
    )(positions, features, global_features, wgp, wgf, wgg, bg2, wlg, bl2)

    nodes = jnp.concatenate([positions, features], axis=1).transpose(0, 2, 1)
    ones = jnp.ones((B, N, 1), jnp.float32)
    nodes_ext = jnp.concatenate([nodes, ones], axis=2)
    xyz = positions.transpose(0, 2, 1)

    grid = (B, N // _TI)
    local_out = pl.pallas_call(
        _local_body,
        grid=grid,
        in_specs=[
            pl.BlockSpec((1, P, N), lambda b, i: (b, 0, 0)),
            pl.BlockSpec((1, _TI, P), lambda b, i: (b, i, 0)),
            pl.BlockSpec((1, N, C + 1), lambda b, i: (b, 0, 0)),
            pl.BlockSpec((1, _TI, C), lambda b, i: (b, i, 0)),
            pl.BlockSpec((1, 1, C), lambda b, i: (b, 0, 0)),
            pl.BlockSpec((C, C), lambda b, i: (0, 0)),
            pl.BlockSpec((C, C), lambda b, i: (0, 0)),
        ],
        out_specs=pl.BlockSpec((1, _TI, C), lambda b, i: (b, i, 0)),
        out_shape=jax.ShapeDtypeStruct((B, N, C), jnp.float32),
    )(positions, xyz, nodes_ext, nodes, gterm.reshape(B, 1, C), wla, wlb)

    positions_new = local_out[:, :, :P].transpose(0, 2, 1)
    features_new = local_out[:, :, P:].transpose(0, 2, 1)
    global_new = g_out[:, :G]
    return (positions_new, features_new, global_new)
